# Initial kernel scaffold; baseline (speedup 1.0000x reference)
import jax
import jax.numpy as jnp
from jax.experimental import pallas as pl


def kernel(x, edge_index, W_conv, b_conv, W1, b1, W2, b2):
    raise NotImplementedError("write your pallas kernel here")


# speedup vs baseline: 17.3847x; 17.3847x over previous
import functools

import jax
import jax.numpy as jnp
from jax import lax
from jax.experimental import pallas as pl
from jax.experimental.pallas import tpu as pltpu
from jax.experimental.pallas import tpu_sc as plsc

N = 10000
E = 160000
D = 256
H = 128
NC = 2
NS = 16
CHUNK = 128
EPAD = 163840
PAD_ROWS = 256
NROWS = N + PAD_ROWS
DEG_ROWS = 10496
DEG_TILE = DEG_ROWS // NS
DEG_OUT = 10112
ROW_TILE = 624


def _deg_body(dst32, deg_part, dstv, onesv, zerov, degsp):
    c = lax.axis_index("c")
    s = lax.axis_index("s")
    w = c * NS + s

    def _fill(ref, n16, val):
        def body(i, _):
            ref[pl.ds(i * 16, 16)] = jnp.full((16,), val, jnp.float32)
            return 0
        lax.fori_loop(0, n16, body, 0)

    _fill(zerov, DEG_TILE // 16, 0.0)
    _fill(onesv, CHUNK // 16, 1.0)
    pltpu.sync_copy(zerov, degsp.at[pl.ds(s * DEG_TILE, DEG_TILE)])
    plsc.subcore_barrier()

    pltpu.sync_copy(dst32.at[w], dstv)

    def edge_chunk(j, _):
        pltpu.sync_copy(onesv, degsp.at[dstv.at[j]], add=True)
        return 0
    lax.fori_loop(0, EPAD // (NC * NS * CHUNK), edge_chunk, 0)

    plsc.subcore_barrier()

    @pl.when(s == 0)
    def _():
        pltpu.sync_copy(degsp.at[pl.ds(0, DEG_OUT)], deg_part.at[c])


def _deg_kernel(dst32):
    mesh = plsc.VectorSubcoreMesh(core_axis_name="c", subcore_axis_name="s")
    f = pl.kernel(
        _deg_body,
        out_type=jax.ShapeDtypeStruct((NC, DEG_OUT), jnp.float32),
        mesh=mesh,
        scratch_types=[
            pltpu.VMEM((EPAD // (NC * NS * CHUNK), CHUNK), jnp.int32),
            pltpu.VMEM((CHUNK,), jnp.float32),
            pltpu.VMEM((DEG_TILE,), jnp.float32),
            pltpu.VMEM_SHARED((DEG_ROWS,), jnp.float32),
        ],
    )
    return f(dst32)


def _agg_body(src16, dst16, h0, h1, agg0, agg1, srcv, dstv, rowsv, aggsp):
    c = lax.axis_index("c")
    s = lax.axis_index("s")

    def run(h_hbm, agg_hbm):
        @pl.when(s < NS - 1)
        def _():
            pltpu.sync_copy(h_hbm.at[pl.ds(s * ROW_TILE, ROW_TILE)],
                            aggsp.at[pl.ds(s * ROW_TILE, ROW_TILE)])

        @pl.when(s == NS - 1)
        def _():
            pltpu.sync_copy(h_hbm.at[pl.ds((NS - 1) * ROW_TILE, N - (NS - 1) * ROW_TILE)],
                            aggsp.at[pl.ds((NS - 1) * ROW_TILE, N - (NS - 1) * ROW_TILE)])

        pltpu.sync_copy(h_hbm.at[pl.ds(s * 16, 16)],
                        aggsp.at[pl.ds(N + s * 16, 16)])
        plsc.subcore_barrier()

        pltpu.sync_copy(src16.at[s], srcv)
        pltpu.sync_copy(dst16.at[s], dstv)

        def edge_chunk(j, _):
            pltpu.sync_copy(h_hbm.at[srcv.at[j]], rowsv)
            pltpu.sync_copy(rowsv, aggsp.at[dstv.at[j]], add=True)
            return 0
        lax.fori_loop(0, EPAD // (NS * CHUNK), edge_chunk, 0)

        plsc.subcore_barrier()

        @pl.when(s < NS - 1)
        def _():
            pltpu.sync_copy(aggsp.at[pl.ds(s * ROW_TILE, ROW_TILE)],
                            agg_hbm.at[pl.ds(s * ROW_TILE, ROW_TILE)])

        @pl.when(s == NS - 1)
        def _():
            pltpu.sync_copy(aggsp.at[pl.ds((NS - 1) * ROW_TILE, N - (NS - 1) * ROW_TILE)],
                            agg_hbm.at[pl.ds((NS - 1) * ROW_TILE, N - (NS - 1) * ROW_TILE)])

    @pl.when(c == 0)
    def _():
        run(h0, agg0)

    @pl.when(c == 1)
    def _():
        run(h1, agg1)


def _agg_kernel(src16, dst16, h0, h1):
    mesh = plsc.VectorSubcoreMesh(core_axis_name="c", subcore_axis_name="s")
    f = pl.kernel(
        _agg_body,
        out_type=(jax.ShapeDtypeStruct((N, H), jnp.float32),
                  jax.ShapeDtypeStruct((N, H), jnp.float32)),
        mesh=mesh,
        scratch_types=[
            pltpu.VMEM((EPAD // (NS * CHUNK), CHUNK), jnp.int32),
            pltpu.VMEM((EPAD // (NS * CHUNK), CHUNK), jnp.int32),
            pltpu.VMEM((CHUNK, H), jnp.float32),
            pltpu.VMEM_SHARED((NROWS, H), jnp.float32),
        ],
    )
    return f(src16, dst16, h0, h1)


BLK = 1000


def _tc_a_body(x_ref, w_ref, b_ref, degt_ref, h0_ref, h1_ref):
    deg = degt_ref[...][:, 0] + degt_ref[...][:, 1] + 1.0
    dinv = lax.rsqrt(deg)
    h = jnp.dot(x_ref[...], w_ref[...], preferred_element_type=jnp.float32)
    h = (h + b_ref[...]) * dinv[:, None]
    h0_ref[...] = h[:, :H]
    h1_ref[...] = h[:, H:]


def _tc_a(x, W_conv, b_conv, deg_t):
    grid = (N // BLK,)
    return pl.pallas_call(
        _tc_a_body,
        grid=grid,
        in_specs=[
            pl.BlockSpec((BLK, D), lambda i: (i, 0)),
            pl.BlockSpec((D, D), lambda i: (0, 0)),
            pl.BlockSpec((1, D), lambda i: (0, 0)),
            pl.BlockSpec((BLK, 2), lambda i: (i, 0)),
        ],
        out_specs=[
            pl.BlockSpec((BLK, H), lambda i: (i, 0)),
            pl.BlockSpec((BLK, H), lambda i: (i, 0)),
        ],
        out_shape=[
            jax.ShapeDtypeStruct((N, H), jnp.float32),
            jax.ShapeDtypeStruct((N, H), jnp.float32),
        ],
    )(x, W_conv, b_conv.reshape(1, D), deg_t)


def _tc_b_body(a0_ref, a1_ref, degt_ref, w1_ref, b1_ref, w2_ref, b2_ref,
               emb_ref, log_ref):
    deg = degt_ref[...][:, 0] + degt_ref[...][:, 1] + 1.0
    dinv = lax.rsqrt(deg)
    agg = jnp.concatenate([a0_ref[...], a1_ref[...]], axis=1) * dinv[:, None]
    emb_ref[...] = agg
    Xr = jnp.maximum(agg, 0.0)
    X = jnp.dot(Xr, w1_ref[...], preferred_element_type=jnp.float32) + b1_ref[...]
    X = jnp.dot(X, w2_ref[...], preferred_element_type=jnp.float32) + b2_ref[...]
    m = jnp.max(X, axis=1, keepdims=True)
    sh = X - m
    lse = jnp.log(jnp.sum(jnp.exp(sh), axis=1, keepdims=True))
    log_ref[...] = sh - lse


def _tc_b(agg0, agg1, deg_t, W1, b1, W2, b2):
    grid = (N // BLK,)
    return pl.pallas_call(
        _tc_b_body,
        grid=grid,
        in_specs=[
            pl.BlockSpec((BLK, H), lambda i: (i, 0)),
            pl.BlockSpec((BLK, H), lambda i: (i, 0)),
            pl.BlockSpec((BLK, 2), lambda i: (i, 0)),
            pl.BlockSpec((D, D), lambda i: (0, 0)),
            pl.BlockSpec((1, D), lambda i: (0, 0)),
            pl.BlockSpec((D, D), lambda i: (0, 0)),
            pl.BlockSpec((1, D), lambda i: (0, 0)),
        ],
        out_specs=[
            pl.BlockSpec((BLK, D), lambda i: (i, 0)),
            pl.BlockSpec((BLK, D), lambda i: (i, 0)),
        ],
        out_shape=[
            jax.ShapeDtypeStruct((N, D), jnp.float32),
            jax.ShapeDtypeStruct((N, D), jnp.float32),
        ],
    )(agg0, agg1, deg_t, W1, b1.reshape(1, D), W2, b2.reshape(1, D))


def kernel(x, edge_index, W_conv, b_conv, W1, b1, W2, b2):
    src = edge_index[0]
    dst = edge_index[1]
    npad = EPAD - E
    ar = jnp.arange(npad, dtype=jnp.int32)
    pad_src = (ar * 97) % N
    pad_dst = N + (ar % PAD_ROWS)
    src_p = jnp.concatenate([src, pad_src])
    dst_p = jnp.concatenate([dst, pad_dst])

    dst32 = dst_p.reshape(NC * NS, EPAD // (NC * NS * CHUNK), CHUNK)
    src16 = src_p.reshape(NS, EPAD // (NS * CHUNK), CHUNK)
    dst16 = dst_p.reshape(NS, EPAD // (NS * CHUNK), CHUNK)

    deg_part = _deg_kernel(dst32)
    deg_t = jnp.transpose(deg_part[:, :N])

    h0, h1 = _tc_a(x, W_conv, b_conv, deg_t)
    agg0, agg1 = _agg_kernel(src16, dst16, h0, h1)
    embedding, logits = _tc_b(agg0, agg1, deg_t, W1, b1, W2, b2)
    return (embedding, logits)


# speedup vs baseline: 21.2717x; 1.2236x over previous
import functools

import jax
import jax.numpy as jnp
from jax import lax
from jax.experimental import pallas as pl
from jax.experimental.pallas import tpu as pltpu
from jax.experimental.pallas import tpu_sc as plsc

N = 10000
E = 160000
D = 256
H = 128
NC = 2
NS = 16
CHUNK = 128
NPH = 2
EPAD = 163840
PAD_ROWS = 256
NROWS = N + PAD_ROWS
DEG_ROWS = 10496
DEG_TILE = DEG_ROWS // NS
DEG_OUT = 10112
ROW_TILE = 624


def _deg_body(dst32, deg_part, dstv, onesv, zerov, degsp):
    c = lax.axis_index("c")
    s = lax.axis_index("s")
    w = c * NS + s

    def _fill(ref, n16, val):
        def body(i, _):
            ref[pl.ds(i * 16, 16)] = jnp.full((16,), val, jnp.float32)
            return 0
        lax.fori_loop(0, n16, body, 0)

    _fill(zerov, DEG_TILE // 16, 0.0)
    _fill(onesv, CHUNK // 16, 1.0)
    pltpu.sync_copy(zerov, degsp.at[pl.ds(s * DEG_TILE, DEG_TILE)])
    plsc.subcore_barrier()

    pltpu.sync_copy(dst32.at[w], dstv)

    def edge_chunk(j, _):
        pltpu.sync_copy(onesv, degsp.at[dstv.at[j]], add=True)
        return 0
    lax.fori_loop(0, EPAD // (NC * NS * CHUNK), edge_chunk, 0)

    plsc.subcore_barrier()

    @pl.when(s == 0)
    def _():
        pltpu.sync_copy(degsp.at[pl.ds(0, DEG_OUT)], deg_part.at[c])


def _deg_kernel(dst32):
    mesh = plsc.VectorSubcoreMesh(core_axis_name="c", subcore_axis_name="s")
    f = pl.kernel(
        _deg_body,
        out_type=jax.ShapeDtypeStruct((NC, DEG_OUT), jnp.float32),
        mesh=mesh,
        scratch_types=[
            pltpu.VMEM((EPAD // (NC * NS * CHUNK), CHUNK), jnp.int32),
            pltpu.VMEM((CHUNK,), jnp.float32),
            pltpu.VMEM((DEG_TILE,), jnp.float32),
            pltpu.VMEM_SHARED((DEG_ROWS,), jnp.float32),
        ],
    )
    return f(dst32)


def _agg_body(src16, dst16, h0, h1, agg0, agg1, srcv, dstv, rowsv, aggsp, sem):
    c = lax.axis_index("c")
    s = lax.axis_index("s")

    def run(h_hbm, agg_hbm):
        @pl.when(s < NS - 1)
        def _():
            pltpu.sync_copy(h_hbm.at[pl.ds(s * ROW_TILE, ROW_TILE)],
                            aggsp.at[pl.ds(s * ROW_TILE, ROW_TILE)])

        @pl.when(s == NS - 1)
        def _():
            pltpu.sync_copy(h_hbm.at[pl.ds((NS - 1) * ROW_TILE, N - (NS - 1) * ROW_TILE)],
                            aggsp.at[pl.ds((NS - 1) * ROW_TILE, N - (NS - 1) * ROW_TILE)])

        pltpu.sync_copy(h_hbm.at[pl.ds(s * 16, 16)],
                        aggsp.at[pl.ds(N + s * 16, 16)])
        plsc.subcore_barrier()

        nch = EPAD // (NS * CHUNK) // NPH

        def phase(k, _):
            pltpu.sync_copy(src16.at[s, pl.ds(k * nch, nch)], srcv)
            pltpu.sync_copy(dst16.at[s, pl.ds(k * nch, nch)], dstv)
            pltpu.async_copy(h_hbm.at[srcv.at[0]], rowsv.at[0], sem)

            def edge_chunk(j, _):
                jm = lax.rem(j, 2)
                pltpu.make_async_copy(h_hbm.at[srcv.at[j]], rowsv.at[jm], sem).wait()

                @pl.when(j + 1 < nch)
                def _():
                    pltpu.async_copy(h_hbm.at[srcv.at[j + 1]], rowsv.at[1 - jm], sem)

                pltpu.sync_copy(rowsv.at[jm], aggsp.at[dstv.at[j]], add=True)
                return 0
            lax.fori_loop(0, nch, edge_chunk, 0)
            return 0
        lax.fori_loop(0, NPH, phase, 0)

        plsc.subcore_barrier()

        @pl.when(s < NS - 1)
        def _():
            pltpu.sync_copy(aggsp.at[pl.ds(s * ROW_TILE, ROW_TILE)],
                            agg_hbm.at[pl.ds(s * ROW_TILE, ROW_TILE)])

        @pl.when(s == NS - 1)
        def _():
            pltpu.sync_copy(aggsp.at[pl.ds((NS - 1) * ROW_TILE, N - (NS - 1) * ROW_TILE)],
                            agg_hbm.at[pl.ds((NS - 1) * ROW_TILE, N - (NS - 1) * ROW_TILE)])

    @pl.when(c == 0)
    def _():
        run(h0, agg0)

    @pl.when(c == 1)
    def _():
        run(h1, agg1)


def _agg_kernel(src16, dst16, h0, h1):
    mesh = plsc.VectorSubcoreMesh(core_axis_name="c", subcore_axis_name="s")
    f = pl.kernel(
        _agg_body,
        out_type=(jax.ShapeDtypeStruct((N, H), jnp.float32),
                  jax.ShapeDtypeStruct((N, H), jnp.float32)),
        mesh=mesh,
        scratch_types=[
            pltpu.VMEM((EPAD // (NS * CHUNK) // NPH, CHUNK), jnp.int32),
            pltpu.VMEM((EPAD // (NS * CHUNK) // NPH, CHUNK), jnp.int32),
            pltpu.VMEM((2, CHUNK, H), jnp.float32),
            pltpu.VMEM_SHARED((NROWS, H), jnp.float32),
            pltpu.SemaphoreType.DMA,
        ],
    )
    return f(src16, dst16, h0, h1)


BLK = 1000


def _tc_a_body(x_ref, w_ref, b_ref, degt_ref, h0_ref, h1_ref):
    deg = degt_ref[...][:, 0] + degt_ref[...][:, 1] + 1.0
    dinv = lax.rsqrt(deg)
    h = jnp.dot(x_ref[...], w_ref[...], preferred_element_type=jnp.float32)
    h = (h + b_ref[...]) * dinv[:, None]
    h0_ref[...] = h[:, :H]
    h1_ref[...] = h[:, H:]


def _tc_a(x, W_conv, b_conv, deg_t):
    grid = (N // BLK,)
    return pl.pallas_call(
        _tc_a_body,
        grid=grid,
        in_specs=[
            pl.BlockSpec((BLK, D), lambda i: (i, 0)),
            pl.BlockSpec((D, D), lambda i: (0, 0)),
            pl.BlockSpec((1, D), lambda i: (0, 0)),
            pl.BlockSpec((BLK, 2), lambda i: (i, 0)),
        ],
        out_specs=[
            pl.BlockSpec((BLK, H), lambda i: (i, 0)),
            pl.BlockSpec((BLK, H), lambda i: (i, 0)),
        ],
        out_shape=[
            jax.ShapeDtypeStruct((N, H), jnp.float32),
            jax.ShapeDtypeStruct((N, H), jnp.float32),
        ],
    )(x, W_conv, b_conv.reshape(1, D), deg_t)


def _tc_b_body(a0_ref, a1_ref, degt_ref, w1_ref, b1_ref, w2_ref, b2_ref,
               emb_ref, log_ref):
    deg = degt_ref[...][:, 0] + degt_ref[...][:, 1] + 1.0
    dinv = lax.rsqrt(deg)
    agg = jnp.concatenate([a0_ref[...], a1_ref[...]], axis=1) * dinv[:, None]
    emb_ref[...] = agg
    Xr = jnp.maximum(agg, 0.0)
    X = jnp.dot(Xr, w1_ref[...], preferred_element_type=jnp.float32) + b1_ref[...]
    X = jnp.dot(X, w2_ref[...], preferred_element_type=jnp.float32) + b2_ref[...]
    m = jnp.max(X, axis=1, keepdims=True)
    sh = X - m
    lse = jnp.log(jnp.sum(jnp.exp(sh), axis=1, keepdims=True))
    log_ref[...] = sh - lse


def _tc_b(agg0, agg1, deg_t, W1, b1, W2, b2):
    grid = (N // BLK,)
    return pl.pallas_call(
        _tc_b_body,
        grid=grid,
        in_specs=[
            pl.BlockSpec((BLK, H), lambda i: (i, 0)),
            pl.BlockSpec((BLK, H), lambda i: (i, 0)),
            pl.BlockSpec((BLK, 2), lambda i: (i, 0)),
            pl.BlockSpec((D, D), lambda i: (0, 0)),
            pl.BlockSpec((1, D), lambda i: (0, 0)),
            pl.BlockSpec((D, D), lambda i: (0, 0)),
            pl.BlockSpec((1, D), lambda i: (0, 0)),
        ],
        out_specs=[
            pl.BlockSpec((BLK, D), lambda i: (i, 0)),
            pl.BlockSpec((BLK, D), lambda i: (i, 0)),
        ],
        out_shape=[
            jax.ShapeDtypeStruct((N, D), jnp.float32),
            jax.ShapeDtypeStruct((N, D), jnp.float32),
        ],
    )(agg0, agg1, deg_t, W1, b1.reshape(1, D), W2, b2.reshape(1, D))


def kernel(x, edge_index, W_conv, b_conv, W1, b1, W2, b2):
    src = edge_index[0]
    dst = edge_index[1]
    npad = EPAD - E
    ar = jnp.arange(npad, dtype=jnp.int32)
    pad_src = (ar * 97) % N
    pad_dst = N + (ar % PAD_ROWS)
    src_p = jnp.concatenate([src, pad_src])
    dst_p = jnp.concatenate([dst, pad_dst])

    dst32 = dst_p.reshape(NC * NS, EPAD // (NC * NS * CHUNK), CHUNK)
    src16 = src_p.reshape(NS, EPAD // (NS * CHUNK), CHUNK)
    dst16 = dst_p.reshape(NS, EPAD // (NS * CHUNK), CHUNK)

    deg_part = _deg_kernel(dst32)
    deg_t = jnp.transpose(deg_part[:, :N])

    h0, h1 = _tc_a(x, W_conv, b_conv, deg_t)
    agg0, agg1 = _agg_kernel(src16, dst16, h0, h1)
    embedding, logits = _tc_b(agg0, agg1, deg_t, W1, b1, W2, b2)
    return (embedding, logits)


# speedup vs baseline: 21.3089x; 1.0017x over previous
import functools

import jax
import jax.numpy as jnp
from jax import lax
from jax.experimental import pallas as pl
from jax.experimental.pallas import tpu as pltpu
from jax.experimental.pallas import tpu_sc as plsc

N = 10000
E = 160000
D = 256
H = 128
NC = 2
NS = 16
CHUNK = 128
NPH = 2
EPAD = 163840
PAD_ROWS = 256
NROWS = N + PAD_ROWS
DEG_ROWS = 10496
DEG_TILE = DEG_ROWS // NS
DEG_OUT = 10112
ROW_TILE = 624


def _deg_body(dst32, deg_part, dstv, onesv, zerov, degsp):
    c = lax.axis_index("c")
    s = lax.axis_index("s")
    w = c * NS + s

    def _fill(ref, n16, val):
        def body(i, _):
            ref[pl.ds(i * 16, 16)] = jnp.full((16,), val, jnp.float32)
            return 0
        lax.fori_loop(0, n16, body, 0)

    _fill(zerov, DEG_TILE // 16, 0.0)
    _fill(onesv, CHUNK // 16, 1.0)
    pltpu.sync_copy(zerov, degsp.at[pl.ds(s * DEG_TILE, DEG_TILE)])
    plsc.subcore_barrier()

    pltpu.sync_copy(dst32.at[w], dstv)

    def edge_chunk(j, _):
        pltpu.sync_copy(onesv, degsp.at[dstv.at[j]], add=True)
        return 0
    lax.fori_loop(0, EPAD // (NC * NS * CHUNK), edge_chunk, 0)

    plsc.subcore_barrier()

    @pl.when(s == 0)
    def _():
        pltpu.sync_copy(degsp.at[pl.ds(0, DEG_OUT)], deg_part.at[c])


def _deg_kernel(dst32):
    mesh = plsc.VectorSubcoreMesh(core_axis_name="c", subcore_axis_name="s")
    f = pl.kernel(
        _deg_body,
        out_type=jax.ShapeDtypeStruct((NC, DEG_OUT), jnp.float32),
        mesh=mesh,
        scratch_types=[
            pltpu.VMEM((EPAD // (NC * NS * CHUNK), CHUNK), jnp.int32),
            pltpu.VMEM((CHUNK,), jnp.float32),
            pltpu.VMEM((DEG_TILE,), jnp.float32),
            pltpu.VMEM_SHARED((DEG_ROWS,), jnp.float32),
        ],
    )
    return f(dst32)


def _agg_body(src16, dst16, h0, h1, agg0, agg1, srcv, dstv, rowsv, aggsp,
              semg, sems):
    c = lax.axis_index("c")
    s = lax.axis_index("s")

    def run(h_hbm, agg_hbm):
        @pl.when(s < NS - 1)
        def _():
            pltpu.sync_copy(h_hbm.at[pl.ds(s * ROW_TILE, ROW_TILE)],
                            aggsp.at[pl.ds(s * ROW_TILE, ROW_TILE)])

        @pl.when(s == NS - 1)
        def _():
            pltpu.sync_copy(h_hbm.at[pl.ds((NS - 1) * ROW_TILE, N - (NS - 1) * ROW_TILE)],
                            aggsp.at[pl.ds((NS - 1) * ROW_TILE, N - (NS - 1) * ROW_TILE)])

        pltpu.sync_copy(h_hbm.at[pl.ds(s * 16, 16)],
                        aggsp.at[pl.ds(N + s * 16, 16)])
        plsc.subcore_barrier()

        nch = EPAD // (NS * CHUNK) // NPH

        def phase(k, _):
            pltpu.sync_copy(src16.at[s, pl.ds(k * nch, nch)], srcv)
            pltpu.sync_copy(dst16.at[s, pl.ds(k * nch, nch)], dstv)
            pltpu.async_copy(h_hbm.at[srcv.at[0]], rowsv.at[0], semg.at[0])

            def edge_chunk(j, _):
                jm = lax.rem(j, 2)
                pltpu.make_async_copy(h_hbm.at[srcv.at[j]], rowsv.at[jm],
                                      semg.at[jm]).wait()

                @pl.when(j + 1 < nch)
                def _():
                    @pl.when(j >= 1)
                    def _():
                        pltpu.make_async_copy(
                            rowsv.at[1 - jm], aggsp.at[dstv.at[j - 1]],
                            sems.at[1 - jm]).wait()
                    pltpu.async_copy(h_hbm.at[srcv.at[j + 1]], rowsv.at[1 - jm],
                                     semg.at[1 - jm])

                pltpu.async_copy(rowsv.at[jm], aggsp.at[dstv.at[j]],
                                 sems.at[jm], add=True)
                return 0
            lax.fori_loop(0, nch, edge_chunk, 0)
            pltpu.make_async_copy(rowsv.at[lax.rem(nch - 2, 2)],
                                  aggsp.at[dstv.at[nch - 2]],
                                  sems.at[lax.rem(nch - 2, 2)]).wait()
            pltpu.make_async_copy(rowsv.at[lax.rem(nch - 1, 2)],
                                  aggsp.at[dstv.at[nch - 1]],
                                  sems.at[lax.rem(nch - 1, 2)]).wait()
            return 0
        lax.fori_loop(0, NPH, phase, 0)

        plsc.subcore_barrier()

        @pl.when(s < NS - 1)
        def _():
            pltpu.sync_copy(aggsp.at[pl.ds(s * ROW_TILE, ROW_TILE)],
                            agg_hbm.at[pl.ds(s * ROW_TILE, ROW_TILE)])

        @pl.when(s == NS - 1)
        def _():
            pltpu.sync_copy(aggsp.at[pl.ds((NS - 1) * ROW_TILE, N - (NS - 1) * ROW_TILE)],
                            agg_hbm.at[pl.ds((NS - 1) * ROW_TILE, N - (NS - 1) * ROW_TILE)])

    @pl.when(c == 0)
    def _():
        run(h0, agg0)

    @pl.when(c == 1)
    def _():
        run(h1, agg1)


def _agg_kernel(src16, dst16, h0, h1):
    mesh = plsc.VectorSubcoreMesh(core_axis_name="c", subcore_axis_name="s")
    f = pl.kernel(
        _agg_body,
        out_type=(jax.ShapeDtypeStruct((N, H), jnp.float32),
                  jax.ShapeDtypeStruct((N, H), jnp.float32)),
        mesh=mesh,
        scratch_types=[
            pltpu.VMEM((EPAD // (NS * CHUNK) // NPH, CHUNK), jnp.int32),
            pltpu.VMEM((EPAD // (NS * CHUNK) // NPH, CHUNK), jnp.int32),
            pltpu.VMEM((2, CHUNK, H), jnp.float32),
            pltpu.VMEM_SHARED((NROWS, H), jnp.float32),
            pltpu.SemaphoreType.DMA((2,)),
            pltpu.SemaphoreType.DMA((2,)),
        ],
    )
    return f(src16, dst16, h0, h1)


BLK = 1000


def _tc_a_body(x_ref, w_ref, b_ref, degt_ref, h0_ref, h1_ref):
    deg = degt_ref[...][:, 0] + degt_ref[...][:, 1] + 1.0
    dinv = lax.rsqrt(deg)
    h = jnp.dot(x_ref[...], w_ref[...], preferred_element_type=jnp.float32)
    h = (h + b_ref[...]) * dinv[:, None]
    h0_ref[...] = h[:, :H]
    h1_ref[...] = h[:, H:]


def _tc_a(x, W_conv, b_conv, deg_t):
    grid = (N // BLK,)
    return pl.pallas_call(
        _tc_a_body,
        grid=grid,
        in_specs=[
            pl.BlockSpec((BLK, D), lambda i: (i, 0)),
            pl.BlockSpec((D, D), lambda i: (0, 0)),
            pl.BlockSpec((1, D), lambda i: (0, 0)),
            pl.BlockSpec((BLK, 2), lambda i: (i, 0)),
        ],
        out_specs=[
            pl.BlockSpec((BLK, H), lambda i: (i, 0)),
            pl.BlockSpec((BLK, H), lambda i: (i, 0)),
        ],
        out_shape=[
            jax.ShapeDtypeStruct((N, H), jnp.float32),
            jax.ShapeDtypeStruct((N, H), jnp.float32),
        ],
    )(x, W_conv, b_conv.reshape(1, D), deg_t)


def _tc_b_body(a0_ref, a1_ref, degt_ref, w1_ref, b1_ref, w2_ref, b2_ref,
               emb_ref, log_ref):
    deg = degt_ref[...][:, 0] + degt_ref[...][:, 1] + 1.0
    dinv = lax.rsqrt(deg)
    agg = jnp.concatenate([a0_ref[...], a1_ref[...]], axis=1) * dinv[:, None]
    emb_ref[...] = agg
    Xr = jnp.maximum(agg, 0.0)
    X = jnp.dot(Xr, w1_ref[...], preferred_element_type=jnp.float32) + b1_ref[...]
    X = jnp.dot(X, w2_ref[...], preferred_element_type=jnp.float32) + b2_ref[...]
    m = jnp.max(X, axis=1, keepdims=True)
    sh = X - m
    lse = jnp.log(jnp.sum(jnp.exp(sh), axis=1, keepdims=True))
    log_ref[...] = sh - lse


def _tc_b(agg0, agg1, deg_t, W1, b1, W2, b2):
    grid = (N // BLK,)
    return pl.pallas_call(
        _tc_b_body,
        grid=grid,
        in_specs=[
            pl.BlockSpec((BLK, H), lambda i: (i, 0)),
            pl.BlockSpec((BLK, H), lambda i: (i, 0)),
            pl.BlockSpec((BLK, 2), lambda i: (i, 0)),
            pl.BlockSpec((D, D), lambda i: (0, 0)),
            pl.BlockSpec((1, D), lambda i: (0, 0)),
            pl.BlockSpec((D, D), lambda i: (0, 0)),
            pl.BlockSpec((1, D), lambda i: (0, 0)),
        ],
        out_specs=[
            pl.BlockSpec((BLK, D), lambda i: (i, 0)),
            pl.BlockSpec((BLK, D), lambda i: (i, 0)),
        ],
        out_shape=[
            jax.ShapeDtypeStruct((N, D), jnp.float32),
            jax.ShapeDtypeStruct((N, D), jnp.float32),
        ],
    )(agg0, agg1, deg_t, W1, b1.reshape(1, D), W2, b2.reshape(1, D))


def kernel(x, edge_index, W_conv, b_conv, W1, b1, W2, b2):
    src = edge_index[0]
    dst = edge_index[1]
    npad = EPAD - E
    ar = jnp.arange(npad, dtype=jnp.int32)
    pad_src = (ar * 97) % N
    pad_dst = N + (ar % PAD_ROWS)
    src_p = jnp.concatenate([src, pad_src])
    dst_p = jnp.concatenate([dst, pad_dst])

    dst32 = dst_p.reshape(NC * NS, EPAD // (NC * NS * CHUNK), CHUNK)
    src16 = src_p.reshape(NS, EPAD // (NS * CHUNK), CHUNK)
    dst16 = dst_p.reshape(NS, EPAD // (NS * CHUNK), CHUNK)

    deg_part = _deg_kernel(dst32)
    deg_t = jnp.transpose(deg_part[:, :N])

    h0, h1 = _tc_a(x, W_conv, b_conv, deg_t)
    agg0, agg1 = _agg_kernel(src16, dst16, h0, h1)
    embedding, logits = _tc_b(agg0, agg1, deg_t, W1, b1, W2, b2)
    return (embedding, logits)


# speedup vs baseline: 24.3091x; 1.1408x over previous
import functools

import jax
import jax.numpy as jnp
from jax import lax
from jax.experimental import pallas as pl
from jax.experimental.pallas import tpu as pltpu
from jax.experimental.pallas import tpu_sc as plsc

N = 10000
E = 160000
D = 256
H = 128
NC = 2
NS = 16
CHUNK = 64
NPH = 4
NBUF = 4
NFLY = 4
EPAD = 163840
PAD_ROWS = 256
NROWS = N + PAD_ROWS
DEG_ROWS = 10496
DEG_TILE = DEG_ROWS // NS
DEG_OUT = 10112
ROW_TILE = 624


def _deg_body(dst32, deg_part, dstv, onesv, zerov, degsp):
    c = lax.axis_index("c")
    s = lax.axis_index("s")
    w = c * NS + s

    def _fill(ref, n16, val):
        def body(i, _):
            ref[pl.ds(i * 16, 16)] = jnp.full((16,), val, jnp.float32)
            return 0
        lax.fori_loop(0, n16, body, 0)

    _fill(zerov, DEG_TILE // 16, 0.0)
    _fill(onesv, CHUNK // 16, 1.0)
    pltpu.sync_copy(zerov, degsp.at[pl.ds(s * DEG_TILE, DEG_TILE)])
    plsc.subcore_barrier()

    pltpu.sync_copy(dst32.at[w], dstv)

    def edge_chunk(j, _):
        pltpu.sync_copy(onesv, degsp.at[dstv.at[j]], add=True)
        return 0
    lax.fori_loop(0, EPAD // (NC * NS * CHUNK), edge_chunk, 0)

    plsc.subcore_barrier()

    @pl.when(s == 0)
    def _():
        pltpu.sync_copy(degsp.at[pl.ds(0, DEG_OUT)], deg_part.at[c])


def _deg_kernel(dst32):
    mesh = plsc.VectorSubcoreMesh(core_axis_name="c", subcore_axis_name="s")
    f = pl.kernel(
        _deg_body,
        out_type=jax.ShapeDtypeStruct((NC, DEG_OUT), jnp.float32),
        mesh=mesh,
        scratch_types=[
            pltpu.VMEM((EPAD // (NC * NS * CHUNK), CHUNK), jnp.int32),
            pltpu.VMEM((CHUNK,), jnp.float32),
            pltpu.VMEM((DEG_TILE,), jnp.float32),
            pltpu.VMEM_SHARED((DEG_ROWS,), jnp.float32),
        ],
    )
    return f(dst32)


def _agg_body(src16, dst16, h0, h1, agg0, agg1, srcv, dstv, rowsv, aggsp,
              semg):
    c = lax.axis_index("c")
    s = lax.axis_index("s")

    def run(h_hbm, agg_hbm):
        @pl.when(s < NS - 1)
        def _():
            pltpu.sync_copy(h_hbm.at[pl.ds(s * ROW_TILE, ROW_TILE)],
                            aggsp.at[pl.ds(s * ROW_TILE, ROW_TILE)])

        @pl.when(s == NS - 1)
        def _():
            pltpu.sync_copy(h_hbm.at[pl.ds((NS - 1) * ROW_TILE, N - (NS - 1) * ROW_TILE)],
                            aggsp.at[pl.ds((NS - 1) * ROW_TILE, N - (NS - 1) * ROW_TILE)])

        pltpu.sync_copy(h_hbm.at[pl.ds(s * 16, 16)],
                        aggsp.at[pl.ds(N + s * 16, 16)])
        plsc.subcore_barrier()

        nch = EPAD // (NS * CHUNK) // NPH

        def phase(k, _):
            pltpu.sync_copy(src16.at[s, pl.ds(k * nch, nch)], srcv)
            pltpu.sync_copy(dst16.at[s, pl.ds(k * nch, nch)], dstv)
            for b in range(NFLY):
                pltpu.async_copy(h_hbm.at[srcv.at[b]], rowsv.at[b], semg.at[b])

            def edge_chunk(j, _):
                jb = lax.rem(j, NBUF)
                pltpu.make_async_copy(h_hbm.at[srcv.at[j]], rowsv.at[jb],
                                      semg.at[jb]).wait()
                pltpu.sync_copy(rowsv.at[jb], aggsp.at[dstv.at[j]], add=True)

                @pl.when(j + NFLY < nch)
                def _():
                    pltpu.async_copy(h_hbm.at[srcv.at[j + NFLY]], rowsv.at[jb],
                                     semg.at[jb])
                return 0
            lax.fori_loop(0, nch, edge_chunk, 0)
            return 0
        lax.fori_loop(0, NPH, phase, 0)

        plsc.subcore_barrier()

        @pl.when(s < NS - 1)
        def _():
            pltpu.sync_copy(aggsp.at[pl.ds(s * ROW_TILE, ROW_TILE)],
                            agg_hbm.at[pl.ds(s * ROW_TILE, ROW_TILE)])

        @pl.when(s == NS - 1)
        def _():
            pltpu.sync_copy(aggsp.at[pl.ds((NS - 1) * ROW_TILE, N - (NS - 1) * ROW_TILE)],
                            agg_hbm.at[pl.ds((NS - 1) * ROW_TILE, N - (NS - 1) * ROW_TILE)])

    @pl.when(c == 0)
    def _():
        run(h0, agg0)

    @pl.when(c == 1)
    def _():
        run(h1, agg1)


def _agg_kernel(src16, dst16, h0, h1):
    mesh = plsc.VectorSubcoreMesh(core_axis_name="c", subcore_axis_name="s")
    f = pl.kernel(
        _agg_body,
        out_type=(jax.ShapeDtypeStruct((N, H), jnp.float32),
                  jax.ShapeDtypeStruct((N, H), jnp.float32)),
        mesh=mesh,
        scratch_types=[
            pltpu.VMEM((EPAD // (NS * CHUNK) // NPH, CHUNK), jnp.int32),
            pltpu.VMEM((EPAD // (NS * CHUNK) // NPH, CHUNK), jnp.int32),
            pltpu.VMEM((NBUF, CHUNK, H), jnp.float32),
            pltpu.VMEM_SHARED((NROWS, H), jnp.float32),
            pltpu.SemaphoreType.DMA((NBUF,)),
        ],
    )
    return f(src16, dst16, h0, h1)


BLK = 1000


def _tc_a_body(x_ref, w_ref, b_ref, degt_ref, h0_ref, h1_ref):
    deg = degt_ref[...][:, 0] + degt_ref[...][:, 1] + 1.0
    dinv = lax.rsqrt(deg)
    h = jnp.dot(x_ref[...], w_ref[...], preferred_element_type=jnp.float32)
    h = (h + b_ref[...]) * dinv[:, None]
    h0_ref[...] = h[:, :H]
    h1_ref[...] = h[:, H:]


def _tc_a(x, W_conv, b_conv, deg_t):
    grid = (N // BLK,)
    return pl.pallas_call(
        _tc_a_body,
        grid=grid,
        in_specs=[
            pl.BlockSpec((BLK, D), lambda i: (i, 0)),
            pl.BlockSpec((D, D), lambda i: (0, 0)),
            pl.BlockSpec((1, D), lambda i: (0, 0)),
            pl.BlockSpec((BLK, 2), lambda i: (i, 0)),
        ],
        out_specs=[
            pl.BlockSpec((BLK, H), lambda i: (i, 0)),
            pl.BlockSpec((BLK, H), lambda i: (i, 0)),
        ],
        out_shape=[
            jax.ShapeDtypeStruct((N, H), jnp.float32),
            jax.ShapeDtypeStruct((N, H), jnp.float32),
        ],
    )(x, W_conv, b_conv.reshape(1, D), deg_t)


def _tc_b_body(a0_ref, a1_ref, degt_ref, w1_ref, b1_ref, w2_ref, b2_ref,
               emb_ref, log_ref):
    deg = degt_ref[...][:, 0] + degt_ref[...][:, 1] + 1.0
    dinv = lax.rsqrt(deg)
    agg = jnp.concatenate([a0_ref[...], a1_ref[...]], axis=1) * dinv[:, None]
    emb_ref[...] = agg
    Xr = jnp.maximum(agg, 0.0)
    X = jnp.dot(Xr, w1_ref[...], preferred_element_type=jnp.float32) + b1_ref[...]
    X = jnp.dot(X, w2_ref[...], preferred_element_type=jnp.float32) + b2_ref[...]
    m = jnp.max(X, axis=1, keepdims=True)
    sh = X - m
    lse = jnp.log(jnp.sum(jnp.exp(sh), axis=1, keepdims=True))
    log_ref[...] = sh - lse


def _tc_b(agg0, agg1, deg_t, W1, b1, W2, b2):
    grid = (N // BLK,)
    return pl.pallas_call(
        _tc_b_body,
        grid=grid,
        in_specs=[
            pl.BlockSpec((BLK, H), lambda i: (i, 0)),
            pl.BlockSpec((BLK, H), lambda i: (i, 0)),
            pl.BlockSpec((BLK, 2), lambda i: (i, 0)),
            pl.BlockSpec((D, D), lambda i: (0, 0)),
            pl.BlockSpec((1, D), lambda i: (0, 0)),
            pl.BlockSpec((D, D), lambda i: (0, 0)),
            pl.BlockSpec((1, D), lambda i: (0, 0)),
        ],
        out_specs=[
            pl.BlockSpec((BLK, D), lambda i: (i, 0)),
            pl.BlockSpec((BLK, D), lambda i: (i, 0)),
        ],
        out_shape=[
            jax.ShapeDtypeStruct((N, D), jnp.float32),
            jax.ShapeDtypeStruct((N, D), jnp.float32),
        ],
    )(agg0, agg1, deg_t, W1, b1.reshape(1, D), W2, b2.reshape(1, D))


def kernel(x, edge_index, W_conv, b_conv, W1, b1, W2, b2):
    src = edge_index[0]
    dst = edge_index[1]
    npad = EPAD - E
    ar = jnp.arange(npad, dtype=jnp.int32)
    pad_src = (ar * 97) % N
    pad_dst = N + (ar % PAD_ROWS)
    src_p = jnp.concatenate([src, pad_src])
    dst_p = jnp.concatenate([dst, pad_dst])

    dst32 = dst_p.reshape(NC * NS, EPAD // (NC * NS * CHUNK), CHUNK)
    src16 = src_p.reshape(NS, EPAD // (NS * CHUNK), CHUNK)
    dst16 = dst_p.reshape(NS, EPAD // (NS * CHUNK), CHUNK)

    deg_part = _deg_kernel(dst32)
    deg_t = jnp.transpose(deg_part[:, :N])

    h0, h1 = _tc_a(x, W_conv, b_conv, deg_t)
    agg0, agg1 = _agg_kernel(src16, dst16, h0, h1)
    embedding, logits = _tc_b(agg0, agg1, deg_t, W1, b1, W2, b2)
    return (embedding, logits)
